# baseline (speedup 1.0000x reference)
import functools

import jax
import jax.numpy as jnp
from jax import lax
from jax.experimental import pallas as pl
from jax.experimental.pallas import tpu as pltpu
from jax.experimental.pallas import tpu_sc as plsc

N = 10000
E = 160000
H = 128
NRBF = 20
LANES = 16

CHUNK = 128
NW = 32
NTILES = 16
EPAD = 163840
NCHPW = EPAD // CHUNK // NW
NPAD = 10112
ROWS_PER_TILE = NPAD // NTILES

BE = 640
NBLK = E // BE
BN = 400


def _mlp_body(rbf_ref, ev_ref, w1_ref, b1_ref, w2_ref, b2_ref,
              fs_ref, f1_ref, t0_ref, t1_ref, t2_ref):
    pid = pl.program_id(0)

    @pl.when(pid < NBLK)
    def _():
        x = jnp.dot(rbf_ref[...], w1_ref[...],
                    preferred_element_type=jnp.float32) + b1_ref[...]
        h = x * jax.nn.sigmoid(x)
        fw = jnp.dot(h, w2_ref[...], preferred_element_type=jnp.float32)
        fw = fw + b2_ref[...]
        fs_ref[...] = fw[:, 0:H]
        f1_ref[...] = fw[:, H:2 * H]
        f2 = fw[:, 2 * H:3 * H]
        t0_ref[...] = f2 * ev_ref[:, 0:1]
        t1_ref[...] = f2 * ev_ref[:, 1:2]
        t2_ref[...] = f2 * ev_ref[:, 2:3]

    @pl.when(pid >= NBLK)
    def _():
        z = jnp.zeros((BE, H), jnp.float32)
        fs_ref[...] = z
        f1_ref[...] = z
        t0_ref[...] = z
        t1_ref[...] = z
        t2_ref[...] = z


def _edge_mlp(rbf, ev, W1, b1, W2, b2):
    out = jax.ShapeDtypeStruct((EPAD, H), jnp.float32)
    return pl.pallas_call(
        _mlp_body,
        grid=(EPAD // BE,),
        in_specs=[
            pl.BlockSpec((BE, NRBF), lambda i: (i, 0)),
            pl.BlockSpec((BE, 3), lambda i: (i, 0)),
            pl.BlockSpec((NRBF, H), lambda i: (0, 0)),
            pl.BlockSpec((1, H), lambda i: (0, 0)),
            pl.BlockSpec((H, 3 * H), lambda i: (0, 0)),
            pl.BlockSpec((1, 3 * H), lambda i: (0, 0)),
        ],
        out_specs=[pl.BlockSpec((BE, H), lambda i: (i, 0))] * 5,
        out_shape=[out, out, out, out, out],
    )(rbf, ev, W1, b1.reshape(1, H), W2, b2.reshape(1, 3 * H))


def _combine_body(sf_ref, vf_ref, p_ref, so_ref, vo_ref):
    so_ref[...] = sf_ref[...] + p_ref[0, 0] + p_ref[0, 1]
    vo_ref[:, 0, :] = vf_ref[:, 0, :] + p_ref[1, 0] + p_ref[1, 1]
    vo_ref[:, 1, :] = vf_ref[:, 1, :] + p_ref[2, 0] + p_ref[2, 1]
    vo_ref[:, 2, :] = vf_ref[:, 2, :] + p_ref[3, 0] + p_ref[3, 1]


def _combine(sf, vf, parts):
    return pl.pallas_call(
        _combine_body,
        grid=(N // BN,),
        in_specs=[
            pl.BlockSpec((BN, H), lambda i: (i, 0)),
            pl.BlockSpec((BN, 3, H), lambda i: (i, 0, 0)),
            pl.BlockSpec((4, 2, BN, H), lambda i: (0, 0, i, 0)),
        ],
        out_specs=[
            pl.BlockSpec((BN, H), lambda i: (i, 0)),
            pl.BlockSpec((BN, 3, H), lambda i: (i, 0, 0)),
        ],
        out_shape=[
            jax.ShapeDtypeStruct((N, H), jnp.float32),
            jax.ShapeDtypeStruct((N, 3, H), jnp.float32),
        ],
    )(sf, vf, parts)


def _make_sc_pass():
    mesh = plsc.VectorSubcoreMesh(core_axis_name="c", subcore_axis_name="s")

    @functools.partial(
        pl.kernel, mesh=mesh,
        out_type=jax.ShapeDtypeStruct((4, 2, NPAD, H), jnp.float32),
        scratch_types=[
            pltpu.VMEM((CHUNK,), jnp.int32),
            pltpu.VMEM((CHUNK,), jnp.int32),
            pltpu.VMEM((CHUNK, H), jnp.float32),
            pltpu.VMEM((CHUNK, H), jnp.float32),
            pltpu.VMEM((CHUNK, H), jnp.float32),
            pltpu.VMEM_SHARED((NPAD, H), jnp.float32),
            pltpu.SemaphoreType.DMA,
        ],
    )
    def sc_pass(sf, v0, v1, v2, srci, dsti, fs, f1, t0, t1, t2, zz, out,
                sall, dall, rows, f1rows, t2rows, acc, sem):
        cid = lax.axis_index("c")
        sid = lax.axis_index("s")
        wid = sid * 2 + cid
        r0 = sid * ROWS_PER_TILE

        phases = ((sf, fs, None), (v0, f1, t0), (v1, f1, t1), (v2, f1, t2))
        for p, (tbl, fil, t2f) in enumerate(phases):
            pltpu.sync_copy(zz.at[pl.ds(r0, ROWS_PER_TILE)],
                            acc.at[pl.ds(r0, ROWS_PER_TILE)])
            plsc.subcore_barrier()

            def chunk_body(i, carry):
                c = wid + i * NW
                c0 = c * CHUNK
                pltpu.sync_copy(srci.at[c], sall)
                pltpu.sync_copy(dsti.at[c], dall)
                cp = pltpu.async_copy(tbl.at[sall], rows, sem)
                pltpu.sync_copy(fil.at[pl.ds(c0, CHUNK)], f1rows)
                if t2f is not None:
                    pltpu.sync_copy(t2f.at[pl.ds(c0, CHUNK)], t2rows)
                cp.wait()

                if t2f is None:
                    def e_body(e, _):
                        for hh in range(H // LANES):
                            s = pl.ds(hh * LANES, LANES)
                            rows[e, s] = rows[e, s] * f1rows[e, s]
                        return 0
                else:
                    def e_body(e, _):
                        for hh in range(H // LANES):
                            s = pl.ds(hh * LANES, LANES)
                            rows[e, s] = (rows[e, s] * f1rows[e, s]
                                          + t2rows[e, s])
                        return 0

                lax.fori_loop(0, CHUNK, e_body, 0)
                pltpu.sync_copy(rows, acc.at[dall], add=True)
                return 0

            lax.fori_loop(0, NCHPW, chunk_body, 0)
            plsc.subcore_barrier()
            pltpu.sync_copy(acc.at[pl.ds(r0, ROWS_PER_TILE)],
                            out.at[p, cid, pl.ds(r0, ROWS_PER_TILE)])

    return sc_pass


_sc_pass = _make_sc_pass()


def kernel(scalar_features, vector_features, edge_index, edge_rbf,
           edge_vector, W1, b1, W2, b2):
    epad = EPAD - E
    src = jnp.pad(edge_index[0], (0, epad))
    dst = jnp.pad(edge_index[1], (0, epad))
    src3 = src.reshape(EPAD // CHUNK, CHUNK)
    dst3 = dst.reshape(EPAD // CHUNK, CHUNK)
    rbf = jnp.pad(edge_rbf, ((0, epad), (0, 0)))
    ev = jnp.pad(edge_vector, ((0, epad), (0, 0)))
    vfT = jnp.transpose(vector_features, (1, 0, 2))

    fs, f1, t0, t1, t2 = _edge_mlp(rbf, ev, W1, b1, W2, b2)
    zz = jnp.zeros((NPAD, H), jnp.float32)

    parts = _sc_pass(scalar_features, vfT[0], vfT[1], vfT[2],
                     src3, dst3, fs, f1, t0, t1, t2, zz)

    return _combine(scalar_features, vector_features, parts)


# speedup vs baseline: 1.3404x; 1.3404x over previous
import functools

import jax
import jax.numpy as jnp
from jax import lax
from jax.experimental import pallas as pl
from jax.experimental.pallas import tpu as pltpu
from jax.experimental.pallas import tpu_sc as plsc

N = 10000
E = 160000
H = 128
NRBF = 20
LANES = 16

CHUNK = 128
NW = 32
NTILES = 16
EPAD = 163840
NCHPW = EPAD // CHUNK // NW
NPAD = 10112
ROWS_PER_TILE = NPAD // NTILES

BE = 640
NBLK = E // BE
BN = 400


def _mlp_body(rbf_ref, ev_ref, w1_ref, b1_ref, w2_ref, b2_ref,
              fs_ref, f1_ref, t0_ref, t1_ref, t2_ref):
    pid = pl.program_id(0)

    @pl.when(pid < NBLK)
    def _():
        x = jnp.dot(rbf_ref[...], w1_ref[...],
                    preferred_element_type=jnp.float32) + b1_ref[...]
        h = x * jax.nn.sigmoid(x)
        fw = jnp.dot(h, w2_ref[...], preferred_element_type=jnp.float32)
        fw = fw + b2_ref[...]
        fs_ref[...] = fw[:, 0:H]
        f1_ref[...] = fw[:, H:2 * H]
        f2 = fw[:, 2 * H:3 * H]
        t0_ref[...] = f2 * ev_ref[:, 0:1]
        t1_ref[...] = f2 * ev_ref[:, 1:2]
        t2_ref[...] = f2 * ev_ref[:, 2:3]

    @pl.when(pid >= NBLK)
    def _():
        z = jnp.zeros((BE, H), jnp.float32)
        fs_ref[...] = z
        f1_ref[...] = z
        t0_ref[...] = z
        t1_ref[...] = z
        t2_ref[...] = z


def _edge_mlp(rbf, ev, W1, b1, W2, b2):
    out = jax.ShapeDtypeStruct((EPAD, H), jnp.float32)
    return pl.pallas_call(
        _mlp_body,
        grid=(EPAD // BE,),
        in_specs=[
            pl.BlockSpec((BE, NRBF), lambda i: (i, 0)),
            pl.BlockSpec((BE, 3), lambda i: (i, 0)),
            pl.BlockSpec((NRBF, H), lambda i: (0, 0)),
            pl.BlockSpec((1, H), lambda i: (0, 0)),
            pl.BlockSpec((H, 3 * H), lambda i: (0, 0)),
            pl.BlockSpec((1, 3 * H), lambda i: (0, 0)),
        ],
        out_specs=[pl.BlockSpec((BE, H), lambda i: (i, 0))] * 5,
        out_shape=[out, out, out, out, out],
    )(rbf, ev, W1, b1.reshape(1, H), W2, b2.reshape(1, 3 * H))


def _combine_body(sf_ref, vf_ref, p_ref, so_ref, vo_ref):
    so_ref[...] = sf_ref[...] + p_ref[0, 0] + p_ref[0, 1]
    vo_ref[:, 0, :] = vf_ref[:, 0, :] + p_ref[1, 0] + p_ref[1, 1]
    vo_ref[:, 1, :] = vf_ref[:, 1, :] + p_ref[2, 0] + p_ref[2, 1]
    vo_ref[:, 2, :] = vf_ref[:, 2, :] + p_ref[3, 0] + p_ref[3, 1]


def _combine(sf, vf, parts):
    return pl.pallas_call(
        _combine_body,
        grid=(N // BN,),
        in_specs=[
            pl.BlockSpec((BN, H), lambda i: (i, 0)),
            pl.BlockSpec((BN, 3, H), lambda i: (i, 0, 0)),
            pl.BlockSpec((4, 2, BN, H), lambda i: (0, 0, i, 0)),
        ],
        out_specs=[
            pl.BlockSpec((BN, H), lambda i: (i, 0)),
            pl.BlockSpec((BN, 3, H), lambda i: (i, 0, 0)),
        ],
        out_shape=[
            jax.ShapeDtypeStruct((N, H), jnp.float32),
            jax.ShapeDtypeStruct((N, 3, H), jnp.float32),
        ],
    )(sf, vf, parts)


def _make_sc_pass():
    mesh = plsc.VectorSubcoreMesh(core_axis_name="c", subcore_axis_name="s")

    @functools.partial(
        pl.kernel, mesh=mesh,
        out_type=jax.ShapeDtypeStruct((4, 2, NPAD, H), jnp.float32),
        scratch_types=[
            pltpu.VMEM((CHUNK,), jnp.int32),
            pltpu.VMEM((CHUNK,), jnp.int32),
            pltpu.VMEM((CHUNK, H), jnp.float32),
            pltpu.VMEM((CHUNK, H), jnp.float32),
            pltpu.VMEM((CHUNK, H), jnp.float32),
            pltpu.VMEM_SHARED((NPAD, H), jnp.float32),
            pltpu.SemaphoreType.DMA,
        ],
    )
    def sc_pass(sf, v0, v1, v2, srci, dsti, fs, f1, t0, t1, t2, zz, out,
                sall, dall, rows, f1rows, t2rows, acc, sem):
        cid = lax.axis_index("c")
        sid = lax.axis_index("s")
        wid = sid * 2 + cid
        r0 = sid * ROWS_PER_TILE

        phases = ((sf, fs, None), (v0, f1, t0), (v1, f1, t1), (v2, f1, t2))
        for p, (tbl, fil, t2f) in enumerate(phases):
            pltpu.sync_copy(zz.at[pl.ds(r0, ROWS_PER_TILE)],
                            acc.at[pl.ds(r0, ROWS_PER_TILE)])
            plsc.subcore_barrier()

            def chunk_body(i, carry):
                c = wid + i * NW
                c0 = c * CHUNK
                pltpu.sync_copy(srci.at[c], sall)
                pltpu.sync_copy(dsti.at[c], dall)
                cp = pltpu.async_copy(tbl.at[sall], rows, sem)
                pltpu.sync_copy(fil.at[pl.ds(c0, CHUNK)], f1rows)
                if t2f is not None:
                    pltpu.sync_copy(t2f.at[pl.ds(c0, CHUNK)], t2rows)
                cp.wait()

                if t2f is None:
                    def e_body(e, _):
                        for hh in range(H // LANES):
                            s = pl.ds(hh * LANES, LANES)
                            rows[e, s] = rows[e, s] * f1rows[e, s]
                        return 0
                else:
                    def e_body(e, _):
                        for hh in range(H // LANES):
                            s = pl.ds(hh * LANES, LANES)
                            rows[e, s] = (rows[e, s] * f1rows[e, s]
                                          + t2rows[e, s])
                        return 0

                lax.fori_loop(0, CHUNK, e_body, 0)
                pltpu.sync_copy(rows, acc.at[dall], add=True)
                return 0

            lax.fori_loop(0, NCHPW, chunk_body, 0)
            plsc.subcore_barrier()
            pltpu.sync_copy(acc.at[pl.ds(r0, ROWS_PER_TILE)],
                            out.at[p, cid, pl.ds(r0, ROWS_PER_TILE)])

    return sc_pass


_sc_pass = _make_sc_pass()


def kernel(scalar_features, vector_features, edge_index, edge_rbf,
           edge_vector, W1, b1, W2, b2):
    epad = EPAD - E
    spread = (jnp.arange(epad, dtype=jnp.int32) * 37) % N
    src = jnp.concatenate([edge_index[0], spread])
    dst = jnp.concatenate([edge_index[1], spread])
    src3 = src.reshape(EPAD // CHUNK, CHUNK)
    dst3 = dst.reshape(EPAD // CHUNK, CHUNK)
    rbf = jnp.pad(edge_rbf, ((0, epad), (0, 0)))
    ev = jnp.pad(edge_vector, ((0, epad), (0, 0)))
    vfT = jnp.transpose(vector_features, (1, 0, 2))

    fs, f1, t0, t1, t2 = _edge_mlp(rbf, ev, W1, b1, W2, b2)
    zz = jnp.zeros((NPAD, H), jnp.float32)

    parts = _sc_pass(scalar_features, vfT[0], vfT[1], vfT[2],
                     src3, dst3, fs, f1, t0, t1, t2, zz)

    return _combine(scalar_features, vector_features, parts)
